# baseline (speedup 1.0000x reference)
from functools import partial

import jax
import jax.numpy as jnp
from jax import lax
from jax.experimental import pallas as pl
from jax.experimental.pallas import tpu as pltpu

_BN_EPS = 1e-5
_PACK = 8
_TILE_ROWS = 2048


def _round_up(v, m):
    return (v + m - 1) // m * m


def _mlp_kernel(y_ref, w1_ref, b1_ref, w2_ref, b2_ref, w3_ref, b3_ref, o_ref):
    y = y_ref[...]
    h1 = jnp.dot(y, w1_ref[...], preferred_element_type=jnp.float32)
    h1 = jnp.maximum(h1 + b1_ref[...], 0.0)
    h2 = jnp.dot(h1, w2_ref[...], preferred_element_type=jnp.float32)
    h2 = jnp.maximum(h2 + b2_ref[...], 0.0)
    o = jnp.dot(h2, w3_ref[...], preferred_element_type=jnp.float32)
    o_ref[...] = o + b3_ref[...]


@jax.jit
def _forward(x, w1, b1, g1, be1, m1, v1, w2, b2, g2, be2, m2, v2, w3, b3):
    n, in_f = x.shape
    h = w1.shape[0]
    h2 = w2.shape[0]
    out_f = w3.shape[0]

    s1 = g1 * lax.rsqrt(v1 + _BN_EPS)
    t1 = be1 - m1 * s1
    s2 = g2 * lax.rsqrt(v2 + _BN_EPS)
    t2 = be2 - m2 * s2

    w2f = w2 * s1[None, :]
    b2f = w2 @ t1 + b2
    w3f = w3 * s2[None, :]
    b3f = w3 @ t2 + b3

    eye = jnp.eye(_PACK, dtype=jnp.float32)
    w1e = jnp.kron(eye, w1.T)
    w2e = jnp.kron(eye, w2f.T)
    w3e = jnp.kron(eye, w3f.T)
    b1e = jnp.tile(b1, _PACK)[None, :]
    b2e = jnp.tile(b2f, _PACK)[None, :]
    b3e = jnp.tile(b3f, _PACK)[None, :]

    rows_per_tile = _PACK * _TILE_ROWS
    n_pad = _round_up(n, rows_per_tile)
    if n_pad != n:
        x = jnp.zeros((n_pad, in_f), x.dtype).at[:n].set(x)
    y = x.reshape(n_pad // _PACK, _PACK * in_f)

    grid = (n_pad // rows_per_tile,)
    const = lambda i: (0, 0)

    cost = pl.CostEstimate(
        flops=2 * (n_pad // _PACK) * (w1e.size + w2e.size + w3e.size),
        transcendentals=0,
        bytes_accessed=n_pad * (in_f + out_f) * 4
        + 4 * (w1e.size + w2e.size + w3e.size),
    )

    out_p = pl.pallas_call(
        _mlp_kernel,
        out_shape=jax.ShapeDtypeStruct((n_pad // _PACK, _PACK * out_f),
                                       jnp.float32),
        grid=grid,
        in_specs=[
            pl.BlockSpec((_TILE_ROWS, _PACK * in_f), lambda i: (i, 0)),
            pl.BlockSpec(w1e.shape, const),
            pl.BlockSpec(b1e.shape, const),
            pl.BlockSpec(w2e.shape, const),
            pl.BlockSpec(b2e.shape, const),
            pl.BlockSpec(w3e.shape, const),
            pl.BlockSpec(b3e.shape, const),
        ],
        out_specs=pl.BlockSpec((_TILE_ROWS, _PACK * out_f), lambda i: (i, 0)),
        compiler_params=pltpu.CompilerParams(
            dimension_semantics=("parallel",),
        ),
        cost_estimate=cost,
    )(y, w1e, b1e, w2e, b2e, w3e, b3e)

    return out_p.reshape(n_pad, out_f)[:n]


def kernel(x, w1, b1, g1, be1, m1, v1, w2, b2, g2, be2, m2, v2, w3, b3):
    return _forward(x, w1, b1, g1, be1, m1, v1, w2, b2, g2, be2, m2, v2,
                    w3, b3)


# speedup vs baseline: 55.9099x; 55.9099x over previous
import jax
import jax.numpy as jnp
from jax import lax
from jax.experimental import pallas as pl
from jax.experimental.pallas import tpu as pltpu

_BN_EPS = 1e-5
_TILE_N = 131072


def _round_up(v, m):
    return (v + m - 1) // m * m


def _mlp_kernel(x_ref, w1t_ref, w2_ref, w3_ref,
                b1_ref, g1_ref, be1_ref, m1_ref, v1_ref,
                b2_ref, g2_ref, be2_ref, m2_ref, v2_ref,
                b3_ref, o_ref):
    s1 = g1_ref[...] * lax.rsqrt(v1_ref[...] + _BN_EPS)
    t1 = be1_ref[...] - m1_ref[...] * s1
    s2 = g2_ref[...] * lax.rsqrt(v2_ref[...] + _BN_EPS)
    t2 = be2_ref[...] - m2_ref[...] * s2

    w2 = w2_ref[...]
    w3 = w3_ref[...]
    w2f = w2 * s1
    w3f = w3 * s2

    def _eye(k):
        r = lax.broadcasted_iota(jnp.int32, (k, k), 0)
        c = lax.broadcasted_iota(jnp.int32, (k, k), 1)
        return (r == c).astype(jnp.float32)

    _colsum = lambda a: jnp.sum(a, axis=1, keepdims=True)
    b1c = _colsum(_eye(32) * b1_ref[...])
    b2c = _colsum(w2 * t1) + _colsum(_eye(16) * b2_ref[...])
    b3c = _colsum(w3 * t2) + _colsum(_eye(2) * b3_ref[...])

    h1 = lax.dot_general(w1t_ref[...], x_ref[...],
                         (((0,), (0,)), ((), ())),
                         preferred_element_type=jnp.float32)
    h1 = jnp.maximum(h1 + b1c, 0.0)
    h2 = jnp.dot(w2f, h1, preferred_element_type=jnp.float32)
    h2 = jnp.maximum(h2 + b2c, 0.0)
    o = jnp.dot(w3f, h2, preferred_element_type=jnp.float32)
    o_ref[...] = o + b3c


@jax.jit
def _forward(x, w1, b1, g1, be1, m1, v1, w2, b2, g2, be2, m2, v2, w3, b3):
    n, in_f = x.shape
    h = w1.shape[0]
    h2 = w2.shape[0]
    out_f = w3.shape[0]

    tn = _round_up(min(_TILE_N, _round_up(n, 128)), 128)
    n_pad = _round_up(n, tn)
    if n_pad == n:
        xt = x.T
    else:
        xt = jnp.zeros((in_f, n_pad), x.dtype).at[:, :n].set(x.T)

    w1t = w1.T
    row = lambda p: p.reshape(1, p.shape[0])

    grid = (n_pad // tn,)
    const = lambda i: (0, 0)

    cost = pl.CostEstimate(
        flops=2 * n_pad * (in_f * h + h * h2 + h2 * out_f),
        transcendentals=0,
        bytes_accessed=n_pad * (in_f + out_f) * 4,
    )

    rows = [row(b1), row(g1), row(be1), row(m1), row(v1),
            row(b2), row(g2), row(be2), row(m2), row(v2), row(b3)]

    out_t = pl.pallas_call(
        _mlp_kernel,
        out_shape=jax.ShapeDtypeStruct((out_f, n_pad), jnp.float32),
        grid=grid,
        in_specs=[pl.BlockSpec((in_f, tn), lambda i: (0, i)),
                  pl.BlockSpec(w1t.shape, const),
                  pl.BlockSpec(w2.shape, const),
                  pl.BlockSpec(w3.shape, const)]
                 + [pl.BlockSpec(r.shape, const) for r in rows],
        out_specs=pl.BlockSpec((out_f, tn), lambda i: (0, i)),
        compiler_params=pltpu.CompilerParams(
            dimension_semantics=("parallel",),
        ),
        cost_estimate=cost,
    )(xt, w1t, w2, w3, *rows)

    return out_t[:, :n].T


def kernel(x, w1, b1, g1, be1, m1, v1, w2, b2, g2, be2, m2, v2, w3, b3):
    return _forward(x, w1, b1, g1, be1, m1, v1, w2, b2, g2, be2, m2, v2,
                    w3, b3)
